# Initial kernel scaffold; baseline (speedup 1.0000x reference)
import jax
import jax.numpy as jnp
from jax.experimental import pallas as pl


def kernel(x, edge_index, batch_size, W0, W1, b1, W2, b2):
    raise NotImplementedError("write your pallas kernel here")


# speedup vs baseline: 3.4632x; 3.4632x over previous
import functools

import jax
import jax.numpy as jnp
from jax import lax
from jax.experimental import pallas as pl
from jax.experimental.pallas import tpu as pltpu
from jax.experimental.pallas import tpu_sc as plsc

N = 10000
D = 128
E = 320000
NC = 2
NS = 16
NW = NC * NS
CHUNK = 128
CH = 80
SL = 16
NSLAB = CH // SL
EP = NW * CH * CHUNK
NPAD = 10240
RPS = NPAD // NS
ZR = 80
BR = 1000

_mesh = plsc.VectorSubcoreMesh(core_axis_name="c", subcore_axis_name="s")


def _deg_body(src_hbm, dst_hbm, out_hbm, idx_v, ones_v, buf_v, dego_s, degi_s):
    c = lax.axis_index("c")
    s = lax.axis_index("s")
    wid = c * NS + s

    def _zfill(i, _):
        buf_v[pl.ds(i * 16, 16)] = jnp.zeros((16,), jnp.float32)
        return 0

    lax.fori_loop(0, RPS // 16, _zfill, 0)
    for i in range(CHUNK // 16):
        ones_v[pl.ds(i * 16, 16)] = jnp.ones((16,), jnp.float32)
    pltpu.sync_copy(buf_v, dego_s.at[pl.ds(s * RPS, RPS)])
    pltpu.sync_copy(buf_v, degi_s.at[pl.ds(s * RPS, RPS)])
    plsc.subcore_barrier()

    pltpu.sync_copy(src_hbm.at[wid], idx_v)

    def _acc_o(j, _):
        pltpu.sync_copy(ones_v, dego_s.at[idx_v.at[j]], add=True)
        return 0

    lax.fori_loop(0, CH, _acc_o, 0)
    pltpu.sync_copy(dst_hbm.at[wid], idx_v)

    def _acc_i(j, _):
        pltpu.sync_copy(ones_v, degi_s.at[idx_v.at[j]], add=True)
        return 0

    lax.fori_loop(0, CH, _acc_i, 0)
    plsc.subcore_barrier()

    pltpu.sync_copy(dego_s.at[pl.ds(s * RPS, RPS)], buf_v)
    pltpu.sync_copy(buf_v, out_hbm.at[c, 0, pl.ds(s * RPS, RPS)])
    pltpu.sync_copy(degi_s.at[pl.ds(s * RPS, RPS)], buf_v)
    pltpu.sync_copy(buf_v, out_hbm.at[c, 1, pl.ds(s * RPS, RPS)])


_deg_call = functools.partial(
    pl.kernel,
    mesh=_mesh,
    out_type=jax.ShapeDtypeStruct((NC, 2, NPAD), jnp.float32),
    scratch_types=[
        pltpu.VMEM((CH, CHUNK), jnp.int32),
        pltpu.VMEM((CHUNK,), jnp.float32),
        pltpu.VMEM((RPS,), jnp.float32),
        pltpu.VMEM_SHARED((NPAD,), jnp.float32),
        pltpu.VMEM_SHARED((NPAD,), jnp.float32),
    ],
)(_deg_body)


def _edge_body(h_hbm, src_hbm, dst_hbm, z_hbm, out_hbm, si_v, di_v, ra_v,
               rb_v, agg_s, semA, semB):
    c = lax.axis_index("c")
    s = lax.axis_index("s")
    wid = c * NS + s
    base = s * RPS

    pltpu.sync_copy(z_hbm, ra_v.at[pl.ds(0, ZR)])
    for k in range(RPS // ZR):
        pltpu.sync_copy(ra_v.at[pl.ds(0, ZR)], agg_s.at[pl.ds(base + k * ZR, ZR)])
    plsc.subcore_barrier()

    def _pair(t, _):
        j = 2 * t
        pltpu.make_async_copy(h_hbm.at[si_v.at[j]], ra_v, semA).wait()
        pltpu.async_copy(h_hbm.at[si_v.at[j + 1]], rb_v, semB)
        pltpu.sync_copy(ra_v, agg_s.at[di_v.at[j]], add=True)
        pltpu.make_async_copy(h_hbm.at[si_v.at[j + 1]], rb_v, semB).wait()
        pltpu.async_copy(h_hbm.at[si_v.at[j + 2]], ra_v, semA)
        pltpu.sync_copy(rb_v, agg_s.at[di_v.at[j + 1]], add=True)
        return 0

    for slab in range(NSLAB):
        pltpu.sync_copy(src_hbm.at[wid, pl.ds(slab * SL, SL)], si_v)
        pltpu.sync_copy(dst_hbm.at[wid, pl.ds(slab * SL, SL)], di_v)
        pltpu.async_copy(h_hbm.at[si_v.at[0]], ra_v, semA)
        lax.fori_loop(0, SL // 2 - 1, _pair, 0)
        j = SL - 2
        pltpu.make_async_copy(h_hbm.at[si_v.at[j]], ra_v, semA).wait()
        pltpu.async_copy(h_hbm.at[si_v.at[j + 1]], rb_v, semB)
        pltpu.sync_copy(ra_v, agg_s.at[di_v.at[j]], add=True)
        pltpu.make_async_copy(h_hbm.at[si_v.at[j + 1]], rb_v, semB).wait()
        pltpu.sync_copy(rb_v, agg_s.at[di_v.at[j + 1]], add=True)

    plsc.subcore_barrier()

    for k in range(RPS // ZR):
        pltpu.sync_copy(agg_s.at[pl.ds(base + k * ZR, ZR)], ra_v.at[pl.ds(0, ZR)])
        pltpu.sync_copy(ra_v.at[pl.ds(0, ZR)], out_hbm.at[c, pl.ds(base + k * ZR, ZR)])


_edge_call = functools.partial(
    pl.kernel,
    mesh=_mesh,
    out_type=jax.ShapeDtypeStruct((NC, NPAD, D), jnp.float32),
    scratch_types=[
        pltpu.VMEM((SL, CHUNK), jnp.int32),
        pltpu.VMEM((SL, CHUNK), jnp.int32),
        pltpu.VMEM((CHUNK, D), jnp.float32),
        pltpu.VMEM((CHUNK, D), jnp.float32),
        pltpu.VMEM_SHARED((NPAD, D), jnp.float32),
        pltpu.SemaphoreType.DMA,
        pltpu.SemaphoreType.DMA,
    ],
)(_edge_body)


def _norm_body(deg_ref, out_ref):
    d = deg_ref[0] + deg_ref[1]
    out_ref[...] = jnp.where(d > 0, lax.rsqrt(jnp.maximum(d, 1e-12)), 0.0)


_norm_call = pl.pallas_call(
    _norm_body,
    out_shape=jax.ShapeDtypeStruct((2, NPAD), jnp.float32),
)


def _mm0_body(x_ref, w_ref, ns_ref, o_ref):
    acc = jnp.dot(x_ref[...], w_ref[...], preferred_element_type=jnp.float32)
    o_ref[...] = acc * ns_ref[...]


_mm0_call = pl.pallas_call(
    _mm0_body,
    grid=(N // BR,),
    in_specs=[
        pl.BlockSpec((BR, D), lambda i: (i, 0)),
        pl.BlockSpec((D, D), lambda i: (0, 0)),
        pl.BlockSpec((BR, 1), lambda i: (i, 0)),
    ],
    out_specs=pl.BlockSpec((BR, D), lambda i: (i, 0)),
    out_shape=jax.ShapeDtypeStruct((N, D), jnp.float32),
)


def _make_layer_call(scale_out):
    def _layer_body(p_ref0, p_ref1, nd_ref, w_ref, b_ref, ns_ref, o_ref):
        agg = (p_ref0[0] + p_ref1[0]) * nd_ref[...]
        r = jnp.dot(agg, w_ref[...], preferred_element_type=jnp.float32)
        r = r + b_ref[...]
        if scale_out:
            r = r * ns_ref[...]
        o_ref[...] = r

    return pl.pallas_call(
        _layer_body,
        grid=(N // BR,),
        in_specs=[
            pl.BlockSpec((1, BR, D), lambda i: (0, i, 0)),
            pl.BlockSpec((1, BR, D), lambda i: (1, i, 0)),
            pl.BlockSpec((BR, 1), lambda i: (i, 0)),
            pl.BlockSpec((D, D), lambda i: (0, 0)),
            pl.BlockSpec((1, D), lambda i: (0, 0)),
            pl.BlockSpec((BR, 1), lambda i: (i, 0)),
        ],
        out_specs=pl.BlockSpec((BR, D), lambda i: (i, 0)),
        out_shape=jax.ShapeDtypeStruct((N, D), jnp.float32),
    )


_layer1_call = _make_layer_call(True)
_layer2_call = _make_layer_call(False)


def kernel(x, edge_index, batch_size, W0, W1, b1, W2, b2):
    src = edge_index[0].astype(jnp.int32)
    dst = edge_index[1].astype(jnp.int32)
    padn = EP - E
    src_g = jnp.concatenate([src, jnp.zeros((padn,), jnp.int32)]).reshape(NW, CH, CHUNK)
    src_d = jnp.concatenate([src, jnp.full((padn,), N, jnp.int32)]).reshape(NW, CH, CHUNK)
    dst_d = jnp.concatenate([dst, jnp.full((padn,), N, jnp.int32)]).reshape(NW, CH, CHUNK)
    zeros_zr = jnp.zeros((ZR, D), jnp.float32)

    deg = _deg_call(src_d, dst_d)
    norms = _norm_call(deg)
    ns = norms[0, :N].reshape(N, 1)
    nd = norms[1, :N].reshape(N, 1)

    h0 = _mm0_call(x, W0, ns)
    p = _edge_call(h0, src_g, dst_d, zeros_zr)
    h1 = _layer1_call(p[:, :N], p[:, :N], nd, W1, b1.reshape(1, D), ns)
    p = _edge_call(h1, src_g, dst_d, zeros_zr)
    h2 = _layer2_call(p[:, :N], p[:, :N], nd, W2, b2.reshape(1, D), ns)
    return h2.reshape(100, N // 100, D)
